# Initial kernel scaffold; baseline (speedup 1.0000x reference)
import jax
import jax.numpy as jnp
from jax.experimental import pallas as pl


def kernel(x, edge_index, edge_attr, edge_params, node_params):
    raise NotImplementedError("write your pallas kernel here")


# speedup vs baseline: 2.2484x; 2.2484x over previous
import functools

import jax
import jax.numpy as jnp
from jax import lax
from jax.experimental import pallas as pl
from jax.experimental.pallas import tpu as pltpu
from jax.experimental.pallas import tpu_sc as plsc

N_NODES = 10000
N_EDGES = 320000
D = 128
L = 16
NC, NS = 2, 16
NW = NC * NS
E_PER_W = N_EDGES // NW
CHUNK = 80
N_CHUNKS = E_PER_W // CHUNK
ROWS_PER_TILE = 632
N_PAD = ROWS_PER_TILE * NS

_f32 = jnp.float32


def _premul_body(x_ref, w1a_ref, w1b_ref, xws_ref, xwr_ref):
    x = x_ref[...]
    xws_ref[...] = jnp.dot(x, w1a_ref[...], preferred_element_type=_f32)
    xwr_ref[...] = jnp.dot(x, w1b_ref[...], preferred_element_type=_f32)


def _premul(x, w1a, w1b):
    blk = 1000
    grid = N_NODES // blk
    return pl.pallas_call(
        _premul_body,
        grid=(grid,),
        in_specs=[
            pl.BlockSpec((blk, D), lambda i: (i, 0)),
            pl.BlockSpec((D, D), lambda i: (0, 0)),
            pl.BlockSpec((D, D), lambda i: (0, 0)),
        ],
        out_specs=[
            pl.BlockSpec((blk, D), lambda i: (i, 0)),
            pl.BlockSpec((blk, D), lambda i: (i, 0)),
        ],
        out_shape=[
            jax.ShapeDtypeStruct((N_NODES, D), _f32),
            jax.ShapeDtypeStruct((N_NODES, D), _f32),
        ],
    )(x, w1a, w1b)


def _gather_body(xws_hbm, xwr_hbm, s_hbm, r_hbm, out_hbm,
                 sidx, ridx, rows_s, rows_r, sem_s, sem_r):
    c = lax.axis_index("c")
    s = lax.axis_index("s")
    wid = s * NC + c
    base = wid * E_PER_W

    def chunk(i, carry):
        off = pl.multiple_of(base + i * CHUNK, 8)
        pltpu.sync_copy(s_hbm.at[pl.ds(off, CHUNK)], sidx)
        pltpu.sync_copy(r_hbm.at[pl.ds(off, CHUNK)], ridx)
        cp1 = pltpu.async_copy(xws_hbm.at[sidx], rows_s, sem_s)
        cp2 = pltpu.async_copy(xwr_hbm.at[ridx], rows_r, sem_r)
        cp1.wait()
        cp2.wait()

        def add_row(j, carry2):
            for k in range(D // L):
                sl = pl.ds(k * L, L)
                rows_s[j, sl] = rows_s[j, sl] + rows_r[j, sl]
            return carry2

        lax.fori_loop(0, CHUNK, add_row, 0)
        pltpu.sync_copy(rows_s, out_hbm.at[pl.ds(off, CHUNK)])
        return carry

    lax.fori_loop(0, N_CHUNKS, chunk, 0)


def _sc_gather(xws, xwr, senders, receivers):
    mesh = plsc.VectorSubcoreMesh(
        core_axis_name="c", subcore_axis_name="s",
        num_cores=NC, num_subcores=NS)
    return pl.kernel(
        _gather_body,
        out_type=jax.ShapeDtypeStruct((N_EDGES, D), _f32),
        mesh=mesh,
        scratch_types=[
            pltpu.VMEM((CHUNK,), jnp.int32),
            pltpu.VMEM((CHUNK,), jnp.int32),
            pltpu.VMEM((CHUNK, D), _f32),
            pltpu.VMEM((CHUNK, D), _f32),
            pltpu.SemaphoreType.DMA,
            pltpu.SemaphoreType.DMA,
        ],
    )(xws, xwr, senders, receivers)


def _edge_mlp_body(g_ref, ea_ref, w1c_ref, b1_ref, w2_ref, b2_ref,
                   w3_ref, b3_ref, w4_ref, b4_ref, gm_ref, bt_ref, out_ref):
    h = g_ref[...] + jnp.dot(ea_ref[...], w1c_ref[...],
                             preferred_element_type=_f32) + b1_ref[...]
    h = jnp.maximum(h, 0.0)
    h = jnp.maximum(jnp.dot(h, w2_ref[...], preferred_element_type=_f32)
                    + b2_ref[...], 0.0)
    h = jnp.maximum(jnp.dot(h, w3_ref[...], preferred_element_type=_f32)
                    + b3_ref[...], 0.0)
    h = jnp.dot(h, w4_ref[...], preferred_element_type=_f32) + b4_ref[...]
    mu = jnp.mean(h, axis=-1, keepdims=True)
    hc = h - mu
    var = jnp.mean(hc * hc, axis=-1, keepdims=True)
    out_ref[...] = hc * lax.rsqrt(var + 1e-5) * gm_ref[...] + bt_ref[...]


def _edge_mlp(g, edge_attr, w1c, b1, w2, b2, w3, b3, w4, b4, gamma, beta):
    blk = 512
    grid = N_EDGES // blk
    row = lambda i: (i, 0)
    const2 = lambda i: (0, 0)
    wspec = pl.BlockSpec((D, D), const2)
    bspec = pl.BlockSpec((1, D), const2)
    return pl.pallas_call(
        _edge_mlp_body,
        grid=(grid,),
        in_specs=[
            pl.BlockSpec((blk, D), row),
            pl.BlockSpec((blk, D), row),
            wspec, bspec, wspec, bspec, wspec, bspec, wspec, bspec,
            bspec, bspec,
        ],
        out_specs=pl.BlockSpec((blk, D), row),
        out_shape=jax.ShapeDtypeStruct((N_EDGES, D), _f32),
    )(g, edge_attr, w1c, b1, w2, b2, w3, b3, w4, b4, gamma, beta)


def _scatter_body(rows_hbm, r_hbm, zeros_hbm, out_hbm, ridx, rows_v, acc):
    c = lax.axis_index("c")
    s = lax.axis_index("s")
    wid = s * NC + c
    base = wid * E_PER_W
    stripe = pl.ds(s * ROWS_PER_TILE, ROWS_PER_TILE)

    pltpu.sync_copy(zeros_hbm.at[stripe], acc.at[stripe])
    plsc.subcore_barrier()

    def chunk(i, carry):
        off = pl.multiple_of(base + i * CHUNK, 8)
        pltpu.sync_copy(r_hbm.at[pl.ds(off, CHUNK)], ridx)
        pltpu.sync_copy(rows_hbm.at[pl.ds(off, CHUNK)], rows_v)
        pltpu.sync_copy(rows_v, acc.at[ridx], add=True)
        return carry

    lax.fori_loop(0, N_CHUNKS, chunk, 0)
    plsc.subcore_barrier()
    pltpu.sync_copy(acc.at[stripe], out_hbm.at[c, stripe])


def _sc_scatter(rows, receivers, zeros):
    mesh = plsc.VectorSubcoreMesh(
        core_axis_name="c", subcore_axis_name="s",
        num_cores=NC, num_subcores=NS)
    return pl.kernel(
        _scatter_body,
        out_type=jax.ShapeDtypeStruct((NC, N_PAD, D), _f32),
        mesh=mesh,
        scratch_types=[
            pltpu.VMEM((CHUNK,), jnp.int32),
            pltpu.VMEM((CHUNK, D), _f32),
            pltpu.VMEM_SHARED((N_PAD, D), _f32),
        ],
    )(rows, receivers, zeros)


def _node_mlp_body(x_ref, p0_ref, p1_ref, v1a_ref, v1b_ref, b1_ref,
                   w2_ref, b2_ref, w3_ref, b3_ref, w4_ref, b4_ref,
                   gm_ref, bt_ref, out_ref):
    agg = p0_ref[...] + p1_ref[...]
    h = (jnp.dot(x_ref[...], v1a_ref[...], preferred_element_type=_f32)
         + jnp.dot(agg, v1b_ref[...], preferred_element_type=_f32)
         + b1_ref[...])
    h = jnp.maximum(h, 0.0)
    h = jnp.maximum(jnp.dot(h, w2_ref[...], preferred_element_type=_f32)
                    + b2_ref[...], 0.0)
    h = jnp.maximum(jnp.dot(h, w3_ref[...], preferred_element_type=_f32)
                    + b3_ref[...], 0.0)
    h = jnp.dot(h, w4_ref[...], preferred_element_type=_f32) + b4_ref[...]
    mu = jnp.mean(h, axis=-1, keepdims=True)
    hc = h - mu
    var = jnp.mean(hc * hc, axis=-1, keepdims=True)
    out_ref[...] = hc * lax.rsqrt(var + 1e-5) * gm_ref[...] + bt_ref[...]


def _node_mlp(x, p0, p1, v1a, v1b, b1, w2, b2, w3, b3, w4, b4, gamma, beta):
    blk = 1000
    grid = N_NODES // blk
    row = lambda i: (i, 0)
    const2 = lambda i: (0, 0)
    wspec = pl.BlockSpec((D, D), const2)
    bspec = pl.BlockSpec((1, D), const2)
    return pl.pallas_call(
        _node_mlp_body,
        grid=(grid,),
        in_specs=[
            pl.BlockSpec((blk, D), row),
            pl.BlockSpec((blk, D), row),
            pl.BlockSpec((blk, D), row),
            wspec, wspec, bspec, wspec, bspec, wspec, bspec, wspec, bspec,
            bspec, bspec,
        ],
        out_specs=pl.BlockSpec((blk, D), row),
        out_shape=jax.ShapeDtypeStruct((N_NODES, D), _f32),
    )(x, p0, p1, v1a, v1b, b1, w2, b2, w3, b3, w4, b4, gamma, beta)


@jax.jit
def _run(x, edge_index, edge_attr, edge_params, node_params):
    W1, b1, W2, b2, W3, b3, W4, b4, g_e, bt_e = edge_params
    V1, c1, V2, c2, V3, c3, V4, c4, g_n, bt_n = node_params
    senders = edge_index[0]
    receivers = edge_index[1]

    w1a, w1b, w1c = W1[:D], W1[D:2 * D], W1[2 * D:]
    xws, xwr = _premul(x, w1a, w1b)
    g = _sc_gather(xws, xwr, senders, receivers)

    r2 = lambda v: v.reshape(1, D)
    eup = _edge_mlp(g, edge_attr, w1c, r2(b1), W2, r2(b2), W3, r2(b3),
                    W4, r2(b4), r2(g_e), r2(bt_e))

    zeros = jnp.zeros((N_PAD, D), _f32)
    partials = _sc_scatter(eup, receivers, zeros)

    x_up = _node_mlp(x, partials[0, :N_NODES], partials[1, :N_NODES],
                     V1[:D], V1[D:], r2(c1),
                     V2, r2(c2), V3, r2(c3), V4, r2(c4), r2(g_n), r2(bt_n))
    return (x_up, eup)


def kernel(x, edge_index, edge_attr, edge_params, node_params):
    return _run(x, edge_index, edge_attr, edge_params, node_params)


# speedup vs baseline: 5.0043x; 2.2257x over previous
import functools

import jax
import jax.numpy as jnp
from jax import lax
from jax.experimental import pallas as pl
from jax.experimental.pallas import tpu as pltpu
from jax.experimental.pallas import tpu_sc as plsc

N_NODES = 10000
N_EDGES = 320000
D = 128
L = 16
NC, NS = 2, 16
NW = NC * NS
E_PER_W = N_EDGES // NW
CHUNK = 80
N_CHUNKS = E_PER_W // CHUNK
N_SLICES = 5
ROWS_PER_TILE = 632
N_PAD = ROWS_PER_TILE * NS

_f32 = jnp.float32


def _premul_body(x_ref, w1a_ref, w1b_ref, xws_ref, xwr_ref):
    x = x_ref[...]
    xws_ref[...] = jnp.dot(x, w1a_ref[...], preferred_element_type=_f32)
    xwr_ref[...] = jnp.dot(x, w1b_ref[...], preferred_element_type=_f32)


def _premul(x, w1a, w1b):
    blk = 1000
    grid = N_NODES // blk
    return pl.pallas_call(
        _premul_body,
        grid=(grid,),
        in_specs=[
            pl.BlockSpec((blk, D), lambda i: (i, 0)),
            pl.BlockSpec((D, D), lambda i: (0, 0)),
            pl.BlockSpec((D, D), lambda i: (0, 0)),
        ],
        out_specs=[
            pl.BlockSpec((blk, D), lambda i: (i, 0)),
            pl.BlockSpec((blk, D), lambda i: (i, 0)),
        ],
        out_shape=[
            jax.ShapeDtypeStruct((N_NODES, D), _f32),
            jax.ShapeDtypeStruct((N_NODES, D), _f32),
        ],
    )(x, w1a, w1b)


def _gather_body(n_edges, xws_hbm, xwr_hbm, s_hbm, r_hbm, out_hbm, *scr):
    bufs = (scr[0:9], scr[9:18])
    e_per_w = n_edges // NW
    n_chunks = e_per_w // CHUNK
    assert n_chunks * CHUNK == e_per_w and n_chunks % 2 == 1
    c = lax.axis_index("c")
    s = lax.axis_index("s")
    wid = s * NC + c
    base = wid * e_per_w

    def off_of(ch):
        return pl.multiple_of(base + ch * CHUNK, 8)

    def idx_start(b, ch):
        sidx, ridx, _, _, sem_is, sem_ir, _, _, _ = bufs[b]
        off = off_of(ch)
        pltpu.async_copy(s_hbm.at[pl.ds(off, CHUNK)], sidx, sem_is)
        pltpu.async_copy(r_hbm.at[pl.ds(off, CHUNK)], ridx, sem_ir)

    def idx_wait(b):
        sidx, ridx, _, _, sem_is, sem_ir, _, _, _ = bufs[b]
        pltpu.make_async_copy(s_hbm.at[pl.ds(0, CHUNK)], sidx, sem_is).wait()
        pltpu.make_async_copy(r_hbm.at[pl.ds(0, CHUNK)], ridx, sem_ir).wait()

    def gather_start(b):
        sidx, ridx, rows_s, rows_r, _, _, sem_gs, sem_gr, _ = bufs[b]
        pltpu.async_copy(xws_hbm.at[sidx], rows_s, sem_gs)
        pltpu.async_copy(xwr_hbm.at[ridx], rows_r, sem_gr)

    def gather_wait(b):
        sidx, ridx, rows_s, rows_r, _, _, sem_gs, sem_gr, _ = bufs[b]
        pltpu.make_async_copy(xws_hbm.at[sidx], rows_s, sem_gs).wait()
        pltpu.make_async_copy(xwr_hbm.at[ridx], rows_r, sem_gr).wait()

    def add(b):
        rows_s, rows_r = bufs[b][2], bufs[b][3]

        def add_row(j, cc):
            for k in range(D // L):
                sl = pl.ds(k * L, L)
                rows_s[j, sl] = rows_s[j, sl] + rows_r[j, sl]
            return cc

        lax.fori_loop(0, CHUNK, add_row, 0)

    def write_start(b, ch):
        rows_s, sem_w = bufs[b][2], bufs[b][8]
        pltpu.async_copy(rows_s, out_hbm.at[pl.ds(off_of(ch), CHUNK)], sem_w)

    def write_wait(b):
        rows_s, sem_w = bufs[b][2], bufs[b][8]
        pltpu.make_async_copy(rows_s, out_hbm.at[pl.ds(0, CHUNK)], sem_w).wait()

    idx_start(0, 0)
    idx_wait(0)
    idx_start(1, 1)
    gather_start(0)

    def pair(p, cc):
        c0 = 2 * p
        c1 = c0 + 1
        c2 = c0 + 2
        c3 = c0 + 3

        @pl.when(c1 < n_chunks)
        def _():
            idx_wait(1)

            @pl.when(p > 0)
            def _():
                write_wait(1)

            gather_start(1)

        gather_wait(0)
        add(0)
        write_start(0, c0)

        @pl.when(c2 < n_chunks)
        def _():
            idx_start(0, c2)

        @pl.when(c1 < n_chunks)
        def _():
            gather_wait(1)
            add(1)
            write_start(1, c1)

            @pl.when(c3 < n_chunks)
            def _():
                idx_start(1, c3)

            @pl.when(c2 < n_chunks)
            def _():
                idx_wait(0)
                write_wait(0)
                gather_start(0)

        return cc

    lax.fori_loop(0, (n_chunks + 1) // 2, pair, 0)
    write_wait(0)
    write_wait(1)


def _sc_gather(xws, xwr, senders, receivers):
    n_edges = senders.shape[0]
    mesh = plsc.VectorSubcoreMesh(
        core_axis_name="c", subcore_axis_name="s",
        num_cores=NC, num_subcores=NS)
    bufset = [
        pltpu.VMEM((CHUNK,), jnp.int32),
        pltpu.VMEM((CHUNK,), jnp.int32),
        pltpu.VMEM((CHUNK, D), _f32),
        pltpu.VMEM((CHUNK, D), _f32),
        pltpu.SemaphoreType.DMA,
        pltpu.SemaphoreType.DMA,
        pltpu.SemaphoreType.DMA,
        pltpu.SemaphoreType.DMA,
        pltpu.SemaphoreType.DMA,
    ]
    return pl.kernel(
        functools.partial(_gather_body, n_edges),
        out_type=jax.ShapeDtypeStruct((n_edges, D), _f32),
        mesh=mesh,
        scratch_types=bufset + bufset,
    )(xws, xwr, senders, receivers)


_bf16 = jnp.bfloat16


def _edge_mlp_body(g_ref, ea_ref, w1c_ref, b1_ref, w2_ref, b2_ref,
                   w3_ref, b3_ref, w4_ref, b4_ref, gm_ref, bt_ref, out_ref):
    h = (g_ref[...]
         + jnp.dot(ea_ref[...].astype(_bf16), w1c_ref[...],
                   preferred_element_type=_f32) + b1_ref[...])
    h = jnp.maximum(h, 0.0)
    h = jnp.maximum(jnp.dot(h.astype(_bf16), w2_ref[...],
                            preferred_element_type=_f32) + b2_ref[...], 0.0)
    h = jnp.maximum(jnp.dot(h.astype(_bf16), w3_ref[...],
                            preferred_element_type=_f32) + b3_ref[...], 0.0)
    h = jnp.dot(h.astype(_bf16), w4_ref[...],
                preferred_element_type=_f32) + b4_ref[...]
    mu = jnp.mean(h, axis=-1, keepdims=True)
    hc = h - mu
    var = jnp.mean(hc * hc, axis=-1, keepdims=True)
    out_ref[...] = hc * lax.rsqrt(var + 1e-5) * gm_ref[...] + bt_ref[...]


def _edge_mlp(g, edge_attr, ea_blk_off, w1c, b1, w2, b2, w3, b3, w4, b4,
              gamma, beta):
    blk = 2560
    n_edges = g.shape[0]
    grid = n_edges // blk
    row = lambda i: (i, 0)
    ea_row = lambda i: (i + ea_blk_off, 0)
    const2 = lambda i: (0, 0)
    wspec = pl.BlockSpec((D, D), const2)
    bspec = pl.BlockSpec((1, D), const2)
    return pl.pallas_call(
        _edge_mlp_body,
        grid=(grid,),
        in_specs=[
            pl.BlockSpec((blk, D), row),
            pl.BlockSpec((blk, D), ea_row),
            wspec, bspec, wspec, bspec, wspec, bspec, wspec, bspec,
            bspec, bspec,
        ],
        out_specs=pl.BlockSpec((blk, D), row),
        out_shape=jax.ShapeDtypeStruct((n_edges, D), _f32),
    )(g, edge_attr, w1c, b1, w2, b2, w3, b3, w4, b4, gamma, beta)


def _scatter_body(n_sl, n_in, k0, *args):
    rows_sl = args[:n_in]
    r_hbm, zeros_hbm, out_hbm, acc = args[n_in:n_in + 4]
    scr = args[n_in + 4:]
    bufs = (scr[0:5], scr[5:10])
    e_per_w_s = n_sl // NW
    n_ch = e_per_w_s // CHUNK
    c = lax.axis_index("c")
    s = lax.axis_index("s")
    wid = s * NC + c
    stripe = pl.ds(s * ROWS_PER_TILE, ROWS_PER_TILE)

    def make_helpers(rows_hbm, k):
        gbase = (k0 + k) * n_sl + wid * e_per_w_s
        lbase = wid * e_per_w_s

        def load_start(b, ch):
            ridx, rows_v, sem_i, sem_l, _ = bufs[b]
            goff = pl.multiple_of(gbase + ch * CHUNK, 8)
            loff = pl.multiple_of(lbase + ch * CHUNK, 8)
            pltpu.async_copy(r_hbm.at[pl.ds(goff, CHUNK)], ridx, sem_i)
            pltpu.async_copy(rows_hbm.at[pl.ds(loff, CHUNK)], rows_v, sem_l)

        def load_wait(b):
            ridx, rows_v, sem_i, sem_l, _ = bufs[b]
            pltpu.make_async_copy(r_hbm.at[pl.ds(0, CHUNK)], ridx,
                                  sem_i).wait()
            pltpu.make_async_copy(rows_hbm.at[pl.ds(0, CHUNK)], rows_v,
                                  sem_l).wait()

        def scatter(b):
            ridx, rows_v, _, _, sem_sc = bufs[b]
            return pltpu.async_copy(rows_v, acc.at[ridx], sem_sc, add=True)

        return load_start, load_wait, scatter

    helpers = [make_helpers(rows_sl[k], k) for k in range(n_in)]

    helpers[0][0](0, 0)
    helpers[0][0](1, 1)
    pltpu.sync_copy(zeros_hbm.at[stripe], acc.at[stripe])
    plsc.subcore_barrier()

    for k in range(n_in):
        load_start, load_wait, scatter = helpers[k]

        def pair(p, cc):
            c0 = 2 * p
            c1 = c0 + 1
            c2 = c0 + 2
            c3 = c0 + 3

            load_wait(0)
            cp0 = scatter(0)

            @pl.when(c1 < n_ch)
            def _():
                load_wait(1)
                scatter(1).wait()

            cp0.wait()

            @pl.when(c2 < n_ch)
            def _():
                load_start(0, c2)

            @pl.when(c3 < n_ch)
            def _():
                load_start(1, c3)

            return cc

        lax.fori_loop(0, (n_ch + 1) // 2, pair, 0)
        if k + 1 < n_in:
            helpers[k + 1][0](0, 0)
            helpers[k + 1][0](1, 1)

    plsc.subcore_barrier()
    pltpu.sync_copy(acc.at[stripe], out_hbm.at[c, stripe])


def _sc_scatter(rows_slices, k0, receivers, zeros):
    n_sl = rows_slices[0].shape[0]
    mesh = plsc.VectorSubcoreMesh(
        core_axis_name="c", subcore_axis_name="s",
        num_cores=NC, num_subcores=NS)
    bufset = [
        pltpu.VMEM((CHUNK,), jnp.int32),
        pltpu.VMEM((CHUNK, D), _f32),
        pltpu.SemaphoreType.DMA,
        pltpu.SemaphoreType.DMA,
        pltpu.SemaphoreType.DMA,
    ]
    return pl.kernel(
        functools.partial(_scatter_body, n_sl, len(rows_slices), k0),
        out_type=jax.ShapeDtypeStruct((NC, N_PAD, D), _f32),
        mesh=mesh,
        scratch_types=[pltpu.VMEM_SHARED((N_PAD, D), _f32)]
        + bufset + bufset,
    )(*rows_slices, receivers, zeros)


def _node_mlp_body(x_ref, p0_ref, p1_ref, p2_ref, p3_ref,
                   v1a_ref, v1b_ref, b1_ref,
                   w2_ref, b2_ref, w3_ref, b3_ref, w4_ref, b4_ref,
                   gm_ref, bt_ref, out_ref):
    agg = ((p0_ref[...] + p1_ref[...]) + (p2_ref[...] + p3_ref[...]))
    h = (jnp.dot(x_ref[...], v1a_ref[...], preferred_element_type=_f32)
         + jnp.dot(agg, v1b_ref[...], preferred_element_type=_f32)
         + b1_ref[...])
    h = jnp.maximum(h, 0.0)
    h = jnp.maximum(jnp.dot(h, w2_ref[...], preferred_element_type=_f32)
                    + b2_ref[...], 0.0)
    h = jnp.maximum(jnp.dot(h, w3_ref[...], preferred_element_type=_f32)
                    + b3_ref[...], 0.0)
    h = jnp.dot(h, w4_ref[...], preferred_element_type=_f32) + b4_ref[...]
    mu = jnp.mean(h, axis=-1, keepdims=True)
    hc = h - mu
    var = jnp.mean(hc * hc, axis=-1, keepdims=True)
    out_ref[...] = hc * lax.rsqrt(var + 1e-5) * gm_ref[...] + bt_ref[...]


def _node_mlp(x, p0, p1, p2, p3, v1a, v1b, b1, w2, b2, w3, b3, w4, b4,
              gamma, beta):
    blk = 1000
    grid = N_NODES // blk
    row = lambda i: (i, 0)
    const2 = lambda i: (0, 0)
    wspec = pl.BlockSpec((D, D), const2)
    bspec = pl.BlockSpec((1, D), const2)
    return pl.pallas_call(
        _node_mlp_body,
        grid=(grid,),
        in_specs=[
            pl.BlockSpec((blk, D), row),
            pl.BlockSpec((blk, D), row),
            pl.BlockSpec((blk, D), row),
            pl.BlockSpec((blk, D), row),
            pl.BlockSpec((blk, D), row),
            wspec, wspec, bspec, wspec, bspec, wspec, bspec, wspec, bspec,
            bspec, bspec,
        ],
        out_specs=pl.BlockSpec((blk, D), row),
        out_shape=jax.ShapeDtypeStruct((N_NODES, D), _f32),
    )(x, p0, p1, p2, p3, v1a, v1b, b1, w2, b2, w3, b3, w4, b4, gamma, beta)


@jax.jit
def _run(x, edge_index, edge_attr, edge_params, node_params):
    W1, b1, W2, b2, W3, b3, W4, b4, g_e, bt_e = edge_params
    V1, c1, V2, c2, V3, c3, V4, c4, g_n, bt_n = node_params
    senders = edge_index[0]
    receivers = edge_index[1]

    w1a, w1b, w1c = W1[:D], W1[D:2 * D], W1[2 * D:]
    xws, xwr = _premul(x, w1a, w1b)
    r2 = lambda v: v.reshape(1, D)
    bf = lambda w: w.astype(jnp.bfloat16)
    edge_w = (bf(w1c), r2(b1), bf(W2), r2(b2), bf(W3), r2(b3), bf(W4),
              r2(b4), r2(g_e), r2(bt_e))

    n_sl = N_EDGES // N_SLICES
    eups = []
    for k in range(N_SLICES):
        sl = slice(k * n_sl, (k + 1) * n_sl)
        g_k = _sc_gather(xws, xwr, senders[sl], receivers[sl])
        eups.append(_edge_mlp(g_k, edge_attr, k * (n_sl // 2560), *edge_w))
    eup = jnp.concatenate(eups, axis=0)

    zeros = jnp.zeros((N_PAD, D), _f32)
    pa = _sc_scatter(eups[:3], 0, receivers, zeros)
    pb = _sc_scatter(eups[3:], 3, receivers, zeros)

    x_up = _node_mlp(x, pa[0, :N_NODES], pa[1, :N_NODES],
                     pb[0, :N_NODES], pb[1, :N_NODES],
                     V1[:D], V1[D:], r2(c1),
                     V2, r2(c2), V3, r2(c3), V4, r2(c4), r2(g_n), r2(bt_n))
    return (x_up, eup)


def kernel(x, edge_index, edge_attr, edge_params, node_params):
    return _run(x, edge_index, edge_attr, edge_params, node_params)
